# baseline (speedup 1.0000x reference)
import functools

import jax
import jax.numpy as jnp
from jax.experimental import pallas as pl
from jax.experimental.pallas import tpu as pltpu


def _body(x_ref, c_ref, m_ref,
          in_w_ref, sw_ref, rs_w_ref, in_b_ref, rs_b_ref,
          o_ref,
          wt_s, ws_s, stt_s, sts_s, rsr_s, rsk_s, b_s,
          *, seqs, chans, stack, taps, dilation_rate):
    G, C, S, K = seqs, chans, stack, taps
    R = G * C
    T = x_ref.shape[-1]

    @pl.when(pl.program_id(0) == 0)
    def _prep():
        p_row = jax.lax.broadcasted_iota(jnp.int32, (R, C), 0) % C
        p_col = jax.lax.broadcasted_iota(jnp.int32, (R, C), 1)
        P = (p_row == p_col).astype(jnp.float32)
        q_row = jax.lax.broadcasted_iota(jnp.int32, (C, R), 0)
        q_col = jax.lax.broadcasted_iota(jnp.int32, (C, R), 1) % C
        Q = (q_row == q_col).astype(jnp.float32)
        blk = (jax.lax.broadcasted_iota(jnp.int32, (R, R), 0) // C ==
               jax.lax.broadcasted_iota(jnp.int32, (R, R), 1) // C)

        def bd(w):
            tiled = jnp.dot(jnp.dot(P, w, preferred_element_type=jnp.float32),
                            Q, preferred_element_type=jnp.float32)
            return jnp.where(blk, tiled, 0.0)

        for i in range(S):
            for k in range(K):
                wt_s[i, k] = bd(in_w_ref[i, k, :C, :])
                ws_s[i, k] = bd(in_w_ref[i, k, C:, :])
            stt_s[i] = bd(sw_ref[i, :C, :])
            sts_s[i] = bd(sw_ref[i, C:, :])
            rsr_s[i] = bd(rs_w_ref[i, :C, :])
            rsk_s[i] = bd(rs_w_ref[i, C:, :])
            b_s[i, 0] = jnp.dot(P, in_b_ref[i, :C, :],
                                preferred_element_type=jnp.float32)
            b_s[i, 1] = jnp.dot(P, in_b_ref[i, C:, :],
                                preferred_element_type=jnp.float32)
            b_s[i, 2] = jnp.dot(P, rs_b_ref[i, :C, :],
                                preferred_element_type=jnp.float32)
            b_s[i, 3] = jnp.dot(P, rs_b_ref[i, C:, :],
                                preferred_element_type=jnp.float32)

    xv = x_ref[...].reshape(R, T)
    cv = c_ref[...].reshape(R, T)
    mask_b = jnp.broadcast_to(m_ref[...], (G, C, T)).reshape(R, T)
    lane = jax.lax.broadcasted_iota(jnp.int32, (R, T), 1)

    skip = jnp.zeros((R, T), jnp.float32)
    for i in range(S):
        d = dilation_rate ** i
        sb_t = (jnp.dot(stt_s[i], cv, preferred_element_type=jnp.float32)
                + b_s[i, 0])
        sb_s = (jnp.dot(sts_s[i], cv, preferred_element_type=jnp.float32)
                + b_s[i, 1])
        tm = jnp.where(lane >= d, pltpu.roll(xv, d, axis=1), 0.0)
        tp = jnp.where(lane < T - d, pltpu.roll(xv, T - d, axis=1), 0.0)
        z_t = (sb_t
               + jnp.dot(wt_s[i, 0], tm, preferred_element_type=jnp.float32)
               + jnp.dot(wt_s[i, 1], xv, preferred_element_type=jnp.float32)
               + jnp.dot(wt_s[i, 2], tp, preferred_element_type=jnp.float32))
        z_s = (sb_s
               + jnp.dot(ws_s[i, 0], tm, preferred_element_type=jnp.float32)
               + jnp.dot(ws_s[i, 1], xv, preferred_element_type=jnp.float32)
               + jnp.dot(ws_s[i, 2], tp, preferred_element_type=jnp.float32))
        acts = jnp.tanh(z_t) * jax.nn.sigmoid(z_s)
        res = (jnp.dot(rsr_s[i], acts, preferred_element_type=jnp.float32)
               + b_s[i, 2])
        skip = (skip
                + jnp.dot(rsk_s[i], acts, preferred_element_type=jnp.float32)
                + b_s[i, 3])
        xv = (xv + res) * mask_b

    o_ref[...] = (skip * mask_b).reshape(G, C, T).astype(o_ref.dtype)


def kernel(x, conditions, float_masks, style_w, in_b, in_w, rs_w, rs_b):
    B, C, T = x.shape
    Cs = conditions.shape[1]
    S, K = in_w.shape[0], in_w.shape[1]
    assert Cs == C
    dilation_rate = 2

    G = 32
    while B % G:
        G //= 2
    R = G * C
    num_blocks = B // G

    sw = style_w.reshape(S, 2 * C, Cs)

    body = functools.partial(
        _body, seqs=G, chans=C, stack=S, taps=K,
        dilation_rate=dilation_rate)

    const = lambda *shape: (shape, lambda b: (0,) * len(shape))

    out = pl.pallas_call(
        body,
        out_shape=jax.ShapeDtypeStruct((B, C, T), jnp.float32),
        grid=(num_blocks,),
        in_specs=[
            pl.BlockSpec((G, C, T), lambda b: (b, 0, 0)),
            pl.BlockSpec((G, C, T), lambda b: (b, 0, 0)),
            pl.BlockSpec((G, 1, T), lambda b: (b, 0, 0)),
            pl.BlockSpec(*const(S, K, 2 * C, C)),
            pl.BlockSpec(*const(S, 2 * C, Cs)),
            pl.BlockSpec(*const(S, 2 * C, C)),
            pl.BlockSpec(*const(S, 2 * C, 1)),
            pl.BlockSpec(*const(S, 2 * C, 1)),
        ],
        out_specs=pl.BlockSpec((G, C, T), lambda b: (b, 0, 0)),
        scratch_shapes=[
            pltpu.VMEM((S, K, R, R), jnp.float32),
            pltpu.VMEM((S, K, R, R), jnp.float32),
            pltpu.VMEM((S, R, R), jnp.float32),
            pltpu.VMEM((S, R, R), jnp.float32),
            pltpu.VMEM((S, R, R), jnp.float32),
            pltpu.VMEM((S, R, R), jnp.float32),
            pltpu.VMEM((S, 4, R, 1), jnp.float32),
        ],
        compiler_params=pltpu.CompilerParams(
            dimension_semantics=("arbitrary",),
            vmem_limit_bytes=56 * 1024 * 1024),
    )(x, conditions, float_masks, in_w, sw, rs_w, in_b, rs_b)

    return out


# speedup vs baseline: 1.1687x; 1.1687x over previous
import functools

import jax
import jax.numpy as jnp
from jax.experimental import pallas as pl
from jax.experimental.pallas import tpu as pltpu


def _body(x_ref, c_ref, m_ref,
          in_w_ref, sw_ref, rs_w_ref, in_b_ref, rs_b_ref,
          o_ref,
          ws_ref, wz_s, wrs_s, bz_s, brs_s,
          *, seqs, chans, stack, taps, dilation_rate):
    G, C, S, K = seqs, chans, stack, taps
    R = G * C
    T = x_ref.shape[-1]

    @pl.when(pl.program_id(0) == 0)
    def _prep():
        p_row = jax.lax.broadcasted_iota(jnp.int32, (R, C), 0) % C
        p_col = jax.lax.broadcasted_iota(jnp.int32, (R, C), 1)
        P = (p_row == p_col).astype(jnp.float32)
        q_row = jax.lax.broadcasted_iota(jnp.int32, (C, R), 0)
        q_col = jax.lax.broadcasted_iota(jnp.int32, (C, R), 1) % C
        Q = (q_row == q_col).astype(jnp.float32)
        blk = (jax.lax.broadcasted_iota(jnp.int32, (R, R), 0) // C ==
               jax.lax.broadcasted_iota(jnp.int32, (R, R), 1) // C)

        def bd(w):
            tiled = jnp.dot(jnp.dot(P, w, preferred_element_type=jnp.float32),
                            Q, preferred_element_type=jnp.float32)
            return jnp.where(blk, tiled, 0.0)

        def tile_b(b):
            return jnp.dot(P, b, preferred_element_type=jnp.float32)

        for i in range(S):
            for h in range(2):
                r0, r1 = h * R, (h + 1) * R
                for k in range(K):
                    wz_s[i, r0:r1, k * R:(k + 1) * R] = bd(
                        in_w_ref[i, k, h * C:(h + 1) * C, :])
                wz_s[i, r0:r1, K * R:(K + 1) * R] = bd(
                    sw_ref[i, h * C:(h + 1) * C, :])
                wrs_s[i, r0:r1, :] = bd(rs_w_ref[i, h * C:(h + 1) * C, :])
                bz_s[i, r0:r1, :] = tile_b(in_b_ref[i, h * C:(h + 1) * C, :])
                brs_s[i, r0:r1, :] = tile_b(rs_b_ref[i, h * C:(h + 1) * C, :])

    xv = x_ref[...].reshape(R, T)
    mask_b = jnp.broadcast_to(m_ref[...], (G, C, T)).reshape(R, T)
    lane = jax.lax.broadcasted_iota(jnp.int32, (R, T), 1)

    ws_ref[3 * R:4 * R, :] = c_ref[...].reshape(R, T)

    skip = jnp.zeros((R, T), jnp.float32)
    for i in range(S):
        d = dilation_rate ** i
        ws_ref[0:R, :] = jnp.where(lane >= d, pltpu.roll(xv, d, axis=1), 0.0)
        ws_ref[R:2 * R, :] = xv
        ws_ref[2 * R:3 * R, :] = jnp.where(
            lane < T - d, pltpu.roll(xv, T - d, axis=1), 0.0)
        z = (jnp.dot(wz_s[i], ws_ref[...], preferred_element_type=jnp.float32)
             + bz_s[i])
        acts = jnp.tanh(z[0:R]) * jax.nn.sigmoid(z[R:2 * R])
        ro = (jnp.dot(wrs_s[i], acts, preferred_element_type=jnp.float32)
              + brs_s[i])
        xv = (xv + ro[0:R]) * mask_b
        skip = skip + ro[R:2 * R]

    o_ref[...] = (skip * mask_b).reshape(G, C, T).astype(o_ref.dtype)


def kernel(x, conditions, float_masks, style_w, in_b, in_w, rs_w, rs_b):
    B, C, T = x.shape
    Cs = conditions.shape[1]
    S, K = in_w.shape[0], in_w.shape[1]
    assert Cs == C
    dilation_rate = 2

    G = 16
    while B % G:
        G //= 2
    R = G * C
    num_blocks = B // G

    sw = style_w.reshape(S, 2 * C, Cs)

    body = functools.partial(
        _body, seqs=G, chans=C, stack=S, taps=K,
        dilation_rate=dilation_rate)

    const = lambda *shape: (shape, lambda b: (0,) * len(shape))

    out = pl.pallas_call(
        body,
        out_shape=jax.ShapeDtypeStruct((B, C, T), jnp.float32),
        grid=(num_blocks,),
        in_specs=[
            pl.BlockSpec((G, C, T), lambda b: (b, 0, 0)),
            pl.BlockSpec((G, C, T), lambda b: (b, 0, 0)),
            pl.BlockSpec((G, 1, T), lambda b: (b, 0, 0)),
            pl.BlockSpec(*const(S, K, 2 * C, C)),
            pl.BlockSpec(*const(S, 2 * C, Cs)),
            pl.BlockSpec(*const(S, 2 * C, C)),
            pl.BlockSpec(*const(S, 2 * C, 1)),
            pl.BlockSpec(*const(S, 2 * C, 1)),
        ],
        out_specs=pl.BlockSpec((G, C, T), lambda b: (b, 0, 0)),
        scratch_shapes=[
            pltpu.VMEM(((K + 1) * R, T), jnp.float32),
            pltpu.VMEM((S, 2 * R, (K + 1) * R), jnp.float32),
            pltpu.VMEM((S, 2 * R, R), jnp.float32),
            pltpu.VMEM((S, 2 * R, 1), jnp.float32),
            pltpu.VMEM((S, 2 * R, 1), jnp.float32),
        ],
        compiler_params=pltpu.CompilerParams(
            dimension_semantics=("arbitrary",),
            vmem_limit_bytes=56 * 1024 * 1024),
    )(x, conditions, float_masks, in_w, sw, rs_w, in_b, rs_b)

    return out


# speedup vs baseline: 1.1694x; 1.0007x over previous
import functools

import jax
import jax.numpy as jnp
from jax.experimental import pallas as pl
from jax.experimental.pallas import tpu as pltpu


def _body(x_ref, c_ref, m_ref,
          in_w_ref, sw_ref, rs_w_ref, in_b_ref, rs_b_ref,
          o_ref,
          ws_ref, wz_s, wrs_s, bz_s, brs_s,
          *, seqs, chans, stack, taps, dilation_rate):
    G, C, S, K = seqs, chans, stack, taps
    R = G * C
    T = x_ref.shape[-1]

    @pl.when(pl.program_id(0) == 0)
    def _prep():
        p_row = jax.lax.broadcasted_iota(jnp.int32, (R, C), 0) % C
        p_col = jax.lax.broadcasted_iota(jnp.int32, (R, C), 1)
        P = (p_row == p_col).astype(jnp.float32)
        q_row = jax.lax.broadcasted_iota(jnp.int32, (C, R), 0)
        q_col = jax.lax.broadcasted_iota(jnp.int32, (C, R), 1) % C
        Q = (q_row == q_col).astype(jnp.float32)
        blk = (jax.lax.broadcasted_iota(jnp.int32, (R, R), 0) // C ==
               jax.lax.broadcasted_iota(jnp.int32, (R, R), 1) // C)

        def bd(w):
            tiled = jnp.dot(jnp.dot(P, w, preferred_element_type=jnp.float32),
                            Q, preferred_element_type=jnp.float32)
            return jnp.where(blk, tiled, 0.0).astype(jnp.bfloat16)

        def tile_b(b):
            return jnp.dot(P, b, preferred_element_type=jnp.float32)

        for i in range(S):
            for h in range(2):
                r0, r1 = h * R, (h + 1) * R
                for k in range(K):
                    wz_s[i, r0:r1, k * R:(k + 1) * R] = bd(
                        in_w_ref[i, k, h * C:(h + 1) * C, :])
                wz_s[i, r0:r1, K * R:(K + 1) * R] = bd(
                    sw_ref[i, h * C:(h + 1) * C, :])
                wrs_s[i, r0:r1, :] = bd(rs_w_ref[i, h * C:(h + 1) * C, :])
                bz_s[i, r0:r1, :] = tile_b(in_b_ref[i, h * C:(h + 1) * C, :])
                brs_s[i, r0:r1, :] = tile_b(rs_b_ref[i, h * C:(h + 1) * C, :])

    xv = x_ref[...].reshape(R, T)
    mask_b = jnp.broadcast_to(m_ref[...], (G, C, T)).reshape(R, T)
    lane = jax.lax.broadcasted_iota(jnp.int32, (R, T), 1)

    ws_ref[3 * R:4 * R, :] = c_ref[...].reshape(R, T).astype(jnp.bfloat16)

    skip = jnp.zeros((R, T), jnp.float32)
    for i in range(S):
        d = dilation_rate ** i
        ws_ref[0:R, :] = jnp.where(
            lane >= d, pltpu.roll(xv, d, axis=1), 0.0).astype(jnp.bfloat16)
        ws_ref[R:2 * R, :] = xv.astype(jnp.bfloat16)
        ws_ref[2 * R:3 * R, :] = jnp.where(
            lane < T - d, pltpu.roll(xv, T - d, axis=1),
            0.0).astype(jnp.bfloat16)
        z = (jnp.dot(wz_s[i], ws_ref[...], preferred_element_type=jnp.float32)
             + bz_s[i])
        acts = (jnp.tanh(z[0:R])
                * jax.nn.sigmoid(z[R:2 * R])).astype(jnp.bfloat16)
        ro = (jnp.dot(wrs_s[i], acts, preferred_element_type=jnp.float32)
              + brs_s[i])
        xv = (xv + ro[0:R]) * mask_b
        skip = skip + ro[R:2 * R]

    o_ref[...] = (skip * mask_b).reshape(G, C, T).astype(o_ref.dtype)


def kernel(x, conditions, float_masks, style_w, in_b, in_w, rs_w, rs_b):
    B, C, T = x.shape
    Cs = conditions.shape[1]
    S, K = in_w.shape[0], in_w.shape[1]
    assert Cs == C
    dilation_rate = 2

    G = 16
    while B % G:
        G //= 2
    R = G * C
    num_blocks = B // G

    sw = style_w.reshape(S, 2 * C, Cs)

    body = functools.partial(
        _body, seqs=G, chans=C, stack=S, taps=K,
        dilation_rate=dilation_rate)

    const = lambda *shape: (shape, lambda b: (0,) * len(shape))

    out = pl.pallas_call(
        body,
        out_shape=jax.ShapeDtypeStruct((B, C, T), jnp.float32),
        grid=(num_blocks,),
        in_specs=[
            pl.BlockSpec((G, C, T), lambda b: (b, 0, 0)),
            pl.BlockSpec((G, C, T), lambda b: (b, 0, 0)),
            pl.BlockSpec((G, 1, T), lambda b: (b, 0, 0)),
            pl.BlockSpec(*const(S, K, 2 * C, C)),
            pl.BlockSpec(*const(S, 2 * C, Cs)),
            pl.BlockSpec(*const(S, 2 * C, C)),
            pl.BlockSpec(*const(S, 2 * C, 1)),
            pl.BlockSpec(*const(S, 2 * C, 1)),
        ],
        out_specs=pl.BlockSpec((G, C, T), lambda b: (b, 0, 0)),
        scratch_shapes=[
            pltpu.VMEM(((K + 1) * R, T), jnp.bfloat16),
            pltpu.VMEM((S, 2 * R, (K + 1) * R), jnp.bfloat16),
            pltpu.VMEM((S, 2 * R, R), jnp.bfloat16),
            pltpu.VMEM((S, 2 * R, 1), jnp.float32),
            pltpu.VMEM((S, 2 * R, 1), jnp.float32),
        ],
        compiler_params=pltpu.CompilerParams(
            dimension_semantics=("arbitrary",),
            vmem_limit_bytes=56 * 1024 * 1024),
    )(x, conditions, float_masks, in_w, sw, rs_w, in_b, rs_b)

    return out
